# baseline (speedup 1.0000x reference)
import functools

import jax
import jax.numpy as jnp
from jax import lax
from jax.experimental import pallas as pl
from jax.experimental.pallas import tpu as pltpu
from jax.experimental.pallas import tpu_sc as plsc

_NC = 2
_NS = 16
_NW = _NC * _NS


def _routing_metadata(router_indices, routing_weights, E, BT, NB):
    TK = router_indices.shape
    N = TK[0] * TK[1]
    NPAD = NB * BT
    CH = 128
    NCH = N // CH
    e_flat = router_indices.reshape(-1).astype(jnp.int32)
    onehot = (e_flat[:, None] == jnp.arange(E, dtype=jnp.int32)[None, :])
    onehot = onehot.astype(jnp.float32)
    oh = onehot.reshape(NCH, CH, E)
    tril = jnp.tril(jnp.ones((CH, CH), jnp.float32))
    within_incl = jnp.einsum("ij,cje->cie", tril, oh,
                             preferred_element_type=jnp.float32)
    totals = within_incl[:, -1, :]
    stril = jnp.tril(jnp.ones((NCH, NCH), jnp.float32), k=-1)
    chunkpref = jnp.dot(stril, totals, preferred_element_type=jnp.float32)

    counts = jnp.sum(totals, axis=0).astype(jnp.int32)
    blk_cnt = (counts + BT - 1) // BT
    blk_end = jnp.cumsum(blk_cnt).astype(jnp.int32)
    total_blocks = blk_end[-1]
    poff_f = ((blk_end - blk_cnt) * BT).astype(jnp.float32)

    sel = within_incl + chunkpref[:, None, :] + poff_f[None, None, :]
    pos = (jnp.sum(oh * sel, axis=-1) - 1.0).reshape(N).astype(jnp.int32)

    bids = jnp.arange(NB, dtype=jnp.int32)
    b_c = jnp.minimum(bids, total_blocks - 1)
    expert_b = jnp.searchsorted(blk_end, b_c, side="right").astype(jnp.int32)
    live_b = (bids < total_blocks).astype(jnp.int32)
    return expert_b, live_b, pos


def _make_dispatch_kernel(T, H, K, NPAD, CH):
    tok_w = T // _NW
    n_ch = tok_w // CH
    mesh = plsc.VectorSubcoreMesh(core_axis_name="c", subcore_axis_name="s")

    @functools.partial(
        pl.kernel,
        out_type=jax.ShapeDtypeStruct((NPAD, H), jnp.float32),
        mesh=mesh,
        scratch_types=[
            pltpu.VMEM((K * n_ch, CH), jnp.int32),
            pltpu.VMEM((CH, H), jnp.float32),
            pltpu.VMEM((CH, H), jnp.float32),
            pltpu.SemaphoreType.DMA,
            pltpu.SemaphoreType.DMA,
            pltpu.SemaphoreType.DMA,
            pltpu.SemaphoreType.DMA,
            pltpu.SemaphoreType.DMA,
            pltpu.SemaphoreType.DMA,
        ],
    )
    def dispatch_k(hs_hbm, pos_hbm, xpad_hbm, idx_v, buf0, buf1, semr0, semr1,
                   sems00, sems01, sems10, sems11):
        wid = lax.axis_index("s") * _NC + lax.axis_index("c")
        base = wid * tok_w
        for k in range(K):
            pltpu.sync_copy(pos_hbm.at[k * _NW + wid],
                            idx_v.at[pl.ds(k * n_ch, n_ch)])
        bufs = (buf0, buf1)
        semr = (semr0, semr1)
        semsc = ((sems00, sems01), (sems10, sems11))
        reads = {}
        scat = {}

        def issue_read(c):
            reads[c] = pltpu.async_copy(
                hs_hbm.at[pl.ds(base + c * CH, CH)], bufs[c % 2], semr[c % 2])

        issue_read(0)
        for c in range(n_ch):
            if c + 1 < n_ch:
                if c >= 1:
                    for cp in scat[c - 1]:
                        cp.wait()
                issue_read(c + 1)
            reads[c].wait()
            scat[c] = [
                pltpu.async_copy(bufs[c % 2],
                                 xpad_hbm.at[idx_v.at[k * n_ch + c]],
                                 semsc[k][c % 2])
                for k in range(K)
            ]
        for c_last in (n_ch - 2, n_ch - 1):
            if c_last >= 0:
                for cp in scat[c_last]:
                    cp.wait()

    return dispatch_k


def _make_combine_kernel(T, H, K, NPAD, CH):
    tok_w = T // _NW
    n_ch = tok_w // CH
    mesh = plsc.VectorSubcoreMesh(core_axis_name="c", subcore_axis_name="s")
    n_vec = H // 16

    @functools.partial(
        pl.kernel,
        out_type=jax.ShapeDtypeStruct((T, H), jnp.float32),
        mesh=mesh,
        scratch_types=[
            pltpu.VMEM((K * n_ch, CH), jnp.int32),
            pltpu.VMEM((K * n_ch, CH, 16), jnp.float32),
            pltpu.VMEM((CH, H), jnp.float32),
            pltpu.VMEM((CH, H), jnp.float32),
            pltpu.VMEM((CH, H), jnp.float32),
            pltpu.VMEM((CH, H), jnp.float32),
            pltpu.SemaphoreType.DMA,
            pltpu.SemaphoreType.DMA,
            pltpu.SemaphoreType.DMA,
            pltpu.SemaphoreType.DMA,
        ],
    )
    def combine_k(yw_hbm, pos_hbm, wb_hbm, out_hbm, idx_v, w_v, acc0, tmp0,
                  acc1, tmp1, semg00, semg01, semg10, semg11):
        wid = lax.axis_index("s") * _NC + lax.axis_index("c")
        base = wid * tok_w
        for k in range(K):
            pltpu.sync_copy(pos_hbm.at[k * _NW + wid],
                            idx_v.at[pl.ds(k * n_ch, n_ch)])
            pltpu.sync_copy(wb_hbm.at[k * _NW + wid],
                            w_v.at[pl.ds(k * n_ch, n_ch)])
        accs = (acc0, acc1)
        tmps = (tmp0, tmp1)
        semg = ((semg00, semg01), (semg10, semg11))
        gathers = {}

        def issue_gather(c):
            s = c % 2
            gathers[c] = [
                pltpu.async_copy(yw_hbm.at[idx_v.at[c]], accs[s], semg[0][s]),
                pltpu.async_copy(yw_hbm.at[idx_v.at[n_ch + c]], tmps[s],
                                 semg[1][s]),
            ]

        issue_gather(0)
        for c in range(n_ch):
            s = c % 2
            if c + 1 < n_ch:
                issue_gather(c + 1)
            for cp in gathers[c]:
                cp.wait()
            acc = accs[s]
            tmp = tmps[s]

            def scale_row(i, _):
                w0 = w_v[c, i, :]
                w1 = w_v[n_ch + c, i, :]
                for v in range(n_vec):
                    sl = pl.ds(v * 16, 16)
                    acc[i, sl] = acc[i, sl] * w0 + tmp[i, sl] * w1
                return 0

            lax.fori_loop(0, CH, scale_row, 0)
            pltpu.sync_copy(acc, out_hbm.at[pl.ds(base + c * CH, CH)])

    return combine_k


def _ffn_body(be_ref, live_ref, x_ref, wg_ref, wu_ref, dp_ref, y_ref, *, I):
    @pl.when(live_ref[pl.program_id(0)] != 0)
    def _():
        x = x_ref[...].astype(jnp.bfloat16)
        gate = jnp.dot(x, wg_ref[0].astype(jnp.bfloat16),
                       preferred_element_type=jnp.float32)
        up = jnp.dot(x, wu_ref[0].astype(jnp.bfloat16),
                     preferred_element_type=jnp.float32)
        inter = (jax.nn.silu(gate) * up).astype(jnp.bfloat16)
        y_ref[...] = jnp.dot(inter, dp_ref[0].astype(jnp.bfloat16),
                             preferred_element_type=jnp.float32)


def kernel(hidden_states, router_indices, routing_weights, gate_up_proj, down_proj):
    B, S, H = hidden_states.shape
    E, _, I2 = gate_up_proj.shape
    I = I2 // 2
    T = B * S
    K = router_indices.shape[1]
    N = T * K

    BT = 128
    NB = N // BT + E
    NB = ((NB + 1) // 2) * 2
    NPAD = NB * BT
    A_CH = 32
    C_CH = 16
    assert K == 2, "combine kernel is specialized for K == 2"

    hs = hidden_states.reshape(T, H)
    expert_b, live_b, pos = _routing_metadata(
        router_indices, routing_weights, E, BT, NB)

    tok_w = T // _NW
    posTK = pos.reshape(T, K)
    pos_kt = jnp.concatenate([posTK[:, k] for k in range(K)])
    pos3d_a = pos_kt.reshape(K * _NW, tok_w // A_CH, A_CH)
    pos3d_c = pos_kt.reshape(K * _NW, tok_w // C_CH, C_CH)
    wb = jnp.broadcast_to(
        jnp.transpose(routing_weights.astype(jnp.float32)).reshape(
            K * _NW, tok_w // C_CH, C_CH, 1),
        (K * _NW, tok_w // C_CH, C_CH, 16))

    x_pad = _make_dispatch_kernel(T, H, K, NPAD, A_CH)(hs, pos3d_a)

    grid_spec = pltpu.PrefetchScalarGridSpec(
        num_scalar_prefetch=2,
        grid=(NB,),
        in_specs=[
            pl.BlockSpec((BT, H), lambda b, be, lv: (b * lv[b], 0)),
            pl.BlockSpec((1, H, I), lambda b, be, lv: (be[b], 0, 0)),
            pl.BlockSpec((1, H, I), lambda b, be, lv: (be[b], 0, 1)),
            pl.BlockSpec((1, I, H), lambda b, be, lv: (be[b], 0, 0)),
        ],
        out_specs=pl.BlockSpec(
            (BT, H), lambda b, be, lv: (b * lv[b] + NB * (1 - lv[b]), 0)),
    )
    yw = pl.pallas_call(
        functools.partial(_ffn_body, I=I),
        grid_spec=grid_spec,
        out_shape=jax.ShapeDtypeStruct((NPAD + BT, H), jnp.float32),
        compiler_params=pltpu.CompilerParams(
            dimension_semantics=("arbitrary",),
        ),
    )(expert_b, live_b, x_pad, gate_up_proj, gate_up_proj, down_proj)

    out = _make_combine_kernel(T, H, K, NPAD, C_CH)(yw, pos3d_c, wb)
    return out.reshape(B, S, H)


# speedup vs baseline: 1.0020x; 1.0020x over previous
import functools

import jax
import jax.numpy as jnp
from jax import lax
from jax.experimental import pallas as pl
from jax.experimental.pallas import tpu as pltpu
from jax.experimental.pallas import tpu_sc as plsc

_NC = 2
_NS = 16
_NW = _NC * _NS


def _routing_metadata(router_indices, routing_weights, E, BT, NB):
    TK = router_indices.shape
    N = TK[0] * TK[1]
    NPAD = NB * BT
    CH = 128
    NCH = N // CH
    e_flat = router_indices.reshape(-1).astype(jnp.int32)
    onehot = (e_flat[:, None] == jnp.arange(E, dtype=jnp.int32)[None, :])
    onehot = onehot.astype(jnp.float32)
    oh = onehot.reshape(NCH, CH, E)
    tril = jnp.tril(jnp.ones((CH, CH), jnp.float32))
    within_incl = jnp.einsum("ij,cje->cie", tril, oh,
                             preferred_element_type=jnp.float32)
    totals = within_incl[:, -1, :]
    stril = jnp.tril(jnp.ones((NCH, NCH), jnp.float32), k=-1)
    chunkpref = jnp.dot(stril, totals, preferred_element_type=jnp.float32)

    counts = jnp.sum(totals, axis=0).astype(jnp.int32)
    blk_cnt = (counts + BT - 1) // BT
    blk_end = jnp.cumsum(blk_cnt).astype(jnp.int32)
    total_blocks = blk_end[-1]
    poff_f = ((blk_end - blk_cnt) * BT).astype(jnp.float32)

    sel = within_incl + chunkpref[:, None, :] + poff_f[None, None, :]
    pos = (jnp.sum(oh * sel, axis=-1) - 1.0).reshape(N).astype(jnp.int32)

    bids = jnp.arange(NB, dtype=jnp.int32)
    b_c = jnp.minimum(bids, total_blocks - 1)
    expert_b = jnp.searchsorted(blk_end, b_c, side="right").astype(jnp.int32)
    live_b = (bids < total_blocks).astype(jnp.int32)
    return expert_b, live_b, pos


def _make_dispatch_kernel(T, H, K, NPAD, CH):
    tok_w = T // _NW
    n_ch = tok_w // CH
    mesh = plsc.VectorSubcoreMesh(core_axis_name="c", subcore_axis_name="s")

    @functools.partial(
        pl.kernel,
        out_type=jax.ShapeDtypeStruct((NPAD, H), jnp.float32),
        mesh=mesh,
        scratch_types=[
            pltpu.VMEM((K * n_ch, CH), jnp.int32),
            pltpu.VMEM((CH, H), jnp.float32),
            pltpu.VMEM((CH, H), jnp.float32),
            pltpu.SemaphoreType.DMA,
            pltpu.SemaphoreType.DMA,
            pltpu.SemaphoreType.DMA,
            pltpu.SemaphoreType.DMA,
            pltpu.SemaphoreType.DMA,
            pltpu.SemaphoreType.DMA,
        ],
    )
    def dispatch_k(hs_hbm, pos_hbm, xpad_hbm, idx_v, buf0, buf1, semr0, semr1,
                   sems00, sems01, sems10, sems11):
        wid = lax.axis_index("s") * _NC + lax.axis_index("c")
        base = wid * tok_w
        for k in range(K):
            pltpu.sync_copy(pos_hbm.at[k * _NW + wid],
                            idx_v.at[pl.ds(k * n_ch, n_ch)])
        bufs = (buf0, buf1)
        semr = (semr0, semr1)
        semsc = ((sems00, sems01), (sems10, sems11))
        reads = {}
        scat = {}

        def issue_read(c):
            reads[c] = pltpu.async_copy(
                hs_hbm.at[pl.ds(base + c * CH, CH)], bufs[c % 2], semr[c % 2])

        issue_read(0)
        for c in range(n_ch):
            if c + 1 < n_ch:
                if c >= 1:
                    for cp in scat[c - 1]:
                        cp.wait()
                issue_read(c + 1)
            reads[c].wait()
            scat[c] = [
                pltpu.async_copy(bufs[c % 2],
                                 xpad_hbm.at[idx_v.at[k * n_ch + c]],
                                 semsc[k][c % 2])
                for k in range(K)
            ]
        for c_last in (n_ch - 2, n_ch - 1):
            if c_last >= 0:
                for cp in scat[c_last]:
                    cp.wait()

    return dispatch_k


def _make_combine_kernel(T, H, K, NPAD, CH):
    tok_w = T // _NW
    n_ch = tok_w // CH
    mesh = plsc.VectorSubcoreMesh(core_axis_name="c", subcore_axis_name="s")
    n_vec = H // 16

    @functools.partial(
        pl.kernel,
        out_type=jax.ShapeDtypeStruct((T, H), jnp.float32),
        mesh=mesh,
        scratch_types=[
            pltpu.VMEM((K * n_ch, CH), jnp.int32),
            pltpu.VMEM((K * n_ch, CH, 16), jnp.float32),
            pltpu.VMEM((CH, H), jnp.float32),
            pltpu.VMEM((CH, H), jnp.float32),
            pltpu.VMEM((CH, H), jnp.float32),
            pltpu.VMEM((CH, H), jnp.float32),
            pltpu.SemaphoreType.DMA,
            pltpu.SemaphoreType.DMA,
            pltpu.SemaphoreType.DMA,
            pltpu.SemaphoreType.DMA,
        ],
    )
    def combine_k(yw_hbm, pos_hbm, wb_hbm, out_hbm, idx_v, w_v, acc0, tmp0,
                  acc1, tmp1, semg00, semg01, semg10, semg11):
        wid = lax.axis_index("s") * _NC + lax.axis_index("c")
        base = wid * tok_w
        for k in range(K):
            pltpu.sync_copy(pos_hbm.at[k * _NW + wid],
                            idx_v.at[pl.ds(k * n_ch, n_ch)])
            pltpu.sync_copy(wb_hbm.at[k * _NW + wid],
                            w_v.at[pl.ds(k * n_ch, n_ch)])
        accs = (acc0, acc1)
        tmps = (tmp0, tmp1)
        semg = ((semg00, semg01), (semg10, semg11))
        gathers = {}

        def issue_gather(c):
            s = c % 2
            gathers[c] = [
                pltpu.async_copy(yw_hbm.at[idx_v.at[c]], accs[s], semg[0][s]),
                pltpu.async_copy(yw_hbm.at[idx_v.at[n_ch + c]], tmps[s],
                                 semg[1][s]),
            ]

        issue_gather(0)
        for c in range(n_ch):
            s = c % 2
            if c + 1 < n_ch:
                issue_gather(c + 1)
            for cp in gathers[c]:
                cp.wait()
            acc = accs[s]
            tmp = tmps[s]

            def scale_row(i, _):
                w0 = w_v[c, i, :]
                w1 = w_v[n_ch + c, i, :]
                for v in range(n_vec):
                    sl = pl.ds(v * 16, 16)
                    acc[i, sl] = acc[i, sl] * w0 + tmp[i, sl] * w1
                return 0

            lax.fori_loop(0, CH, scale_row, 0)
            pltpu.sync_copy(acc, out_hbm.at[pl.ds(base + c * CH, CH)])

    return combine_k


def _ffn_body(be_ref, live_ref, x_ref, wg_ref, wu_ref, dp_ref, y_ref, *, I):
    @pl.when(live_ref[pl.program_id(0)] != 0)
    def _():
        x = x_ref[...].astype(jnp.bfloat16)
        gate = jnp.dot(x, wg_ref[0].astype(jnp.bfloat16),
                       preferred_element_type=jnp.float32)
        up = jnp.dot(x, wu_ref[0].astype(jnp.bfloat16),
                     preferred_element_type=jnp.float32)
        inter = (jax.nn.silu(gate) * up).astype(jnp.bfloat16)
        y_ref[...] = jnp.dot(inter, dp_ref[0].astype(jnp.bfloat16),
                             preferred_element_type=jnp.float32)


def kernel(hidden_states, router_indices, routing_weights, gate_up_proj, down_proj):
    B, S, H = hidden_states.shape
    E, _, I2 = gate_up_proj.shape
    I = I2 // 2
    T = B * S
    K = router_indices.shape[1]
    N = T * K

    BT = 128
    NB = N // BT + E
    NB = ((NB + 1) // 2) * 2
    NPAD = NB * BT
    A_CH = 64
    C_CH = 16
    assert K == 2, "combine kernel is specialized for K == 2"

    hs = hidden_states.reshape(T, H)
    expert_b, live_b, pos = _routing_metadata(
        router_indices, routing_weights, E, BT, NB)

    tok_w = T // _NW
    posTK = pos.reshape(T, K)
    pos_kt = jnp.concatenate([posTK[:, k] for k in range(K)])
    pos3d_a = pos_kt.reshape(K * _NW, tok_w // A_CH, A_CH)
    pos3d_c = pos_kt.reshape(K * _NW, tok_w // C_CH, C_CH)
    wb = jnp.broadcast_to(
        jnp.transpose(routing_weights.astype(jnp.float32)).reshape(
            K * _NW, tok_w // C_CH, C_CH, 1),
        (K * _NW, tok_w // C_CH, C_CH, 16))

    x_pad = _make_dispatch_kernel(T, H, K, NPAD, A_CH)(hs, pos3d_a)

    grid_spec = pltpu.PrefetchScalarGridSpec(
        num_scalar_prefetch=2,
        grid=(NB,),
        in_specs=[
            pl.BlockSpec((BT, H), lambda b, be, lv: (b * lv[b], 0)),
            pl.BlockSpec((1, H, I), lambda b, be, lv: (be[b], 0, 0)),
            pl.BlockSpec((1, H, I), lambda b, be, lv: (be[b], 0, 1)),
            pl.BlockSpec((1, I, H), lambda b, be, lv: (be[b], 0, 0)),
        ],
        out_specs=pl.BlockSpec(
            (BT, H), lambda b, be, lv: (b * lv[b] + NB * (1 - lv[b]), 0)),
    )
    yw = pl.pallas_call(
        functools.partial(_ffn_body, I=I),
        grid_spec=grid_spec,
        out_shape=jax.ShapeDtypeStruct((NPAD + BT, H), jnp.float32),
        compiler_params=pltpu.CompilerParams(
            dimension_semantics=("arbitrary",),
        ),
    )(expert_b, live_b, x_pad, gate_up_proj, gate_up_proj, down_proj)

    out = _make_combine_kernel(T, H, K, NPAD, C_CH)(yw, pos3d_c, wb)
    return out.reshape(B, S, H)
